# baseline (speedup 1.0000x reference)
import functools

import jax
import jax.numpy as jnp
from jax import lax
from jax.experimental import pallas as pl
from jax.experimental.pallas import tpu as pltpu
from jax.experimental.pallas import tpu_sc as plsc

BATCH = 16384
N_FIELDS = 26
VOCAB = 100000
EMBED_DIM = 32

TOTAL_ROWS = BATCH * N_FIELDS
NUM_WORKERS = 32
ROWS_PER_W = TOTAL_ROWS // NUM_WORKERS
GROUP = 128
GROUPS_PER_W = ROWS_PER_W // GROUP
CHUNK_GROUPS = 8
CHUNK_ROWS = CHUNK_GROUPS * GROUP
N_CHUNKS = GROUPS_PER_W // CHUNK_GROUPS
LANES = 16


def _emb_body(idx_hbm, tab_hbm, out_hbm, idx_v, rows_v, gsem):
    cid = lax.axis_index("c")
    sid = lax.axis_index("s")
    wid = sid * 2 + cid
    gbase = wid * GROUPS_PER_W

    pltpu.sync_copy(idx_hbm.at[pl.ds(gbase, GROUPS_PER_W)], idx_v)

    lanes = lax.iota(jnp.int32, LANES)

    def off_body(g, carry):
        pos0 = (gbase + g) * GROUP
        for k in range(GROUP // LANES):
            pos = pos0 + k * LANES + lanes
            fld = lax.rem(pos, jnp.int32(N_FIELDS))
            idx_v[g, pl.ds(k * LANES, LANES)] = (
                idx_v[g, pl.ds(k * LANES, LANES)] + fld * VOCAB
            )
        return carry

    lax.fori_loop(0, GROUPS_PER_W, off_body, 0)

    def chunk_body(ci, carry):
        g0 = ci * CHUNK_GROUPS
        copies = []
        for j in range(CHUNK_GROUPS):
            copies.append(
                pltpu.async_copy(
                    tab_hbm.at[idx_v.at[g0 + j]],
                    rows_v.at[pl.ds(j * GROUP, GROUP)],
                    gsem,
                )
            )
        for cp in copies:
            cp.wait()
        out0 = wid * ROWS_PER_W + ci * CHUNK_ROWS
        pltpu.sync_copy(rows_v, out_hbm.at[pl.ds(out0, CHUNK_ROWS)])
        return carry

    lax.fori_loop(0, N_CHUNKS, chunk_body, 0)


_emb_call = pl.kernel(
    _emb_body,
    mesh=plsc.VectorSubcoreMesh(core_axis_name="c", subcore_axis_name="s"),
    out_type=jax.ShapeDtypeStruct((TOTAL_ROWS, EMBED_DIM), jnp.float32),
    scratch_types=[
        pltpu.VMEM((GROUPS_PER_W, GROUP), jnp.int32),
        pltpu.VMEM((CHUNK_ROWS, EMBED_DIM), jnp.float32),
        pltpu.SemaphoreType.DMA,
    ],
    compiler_params=pltpu.CompilerParams(use_tc_tiling_on_sc=False),
)


@jax.jit
def kernel(indices, tables):
    idx2d = indices.astype(jnp.int32).reshape(TOTAL_ROWS // GROUP, GROUP)
    tab = tables.reshape(N_FIELDS * VOCAB, EMBED_DIM)
    out = _emb_call(idx2d, tab)
    return out.reshape(BATCH, N_FIELDS, EMBED_DIM)


# speedup vs baseline: 3.8207x; 3.8207x over previous
import jax
import jax.numpy as jnp
from jax import lax
from jax.experimental import pallas as pl
from jax.experimental.pallas import tpu as pltpu
from jax.experimental.pallas import tpu_sc as plsc

BATCH = 16384
N_FIELDS = 26
VOCAB = 100000
EMBED_DIM = 32

NUM_PAIRS = N_FIELDS * EMBED_DIM
NUM_WORKERS = 32
PAIRS_PER_W = NUM_PAIRS // NUM_WORKERS
BCHUNK = 8192
N_BCHUNKS = BATCH // BCHUNK
LANES = 16


def _emb_body(idx_hbm, tab_hbm, out_hbm, row_v, idx_v, out_v):
    cid = lax.axis_index("c")
    sid = lax.axis_index("s")
    wid = sid * 2 + cid

    def pair_body(j, carry):
        p = wid * PAIRS_PER_W + j
        f = p // EMBED_DIM
        d = p % EMBED_DIM
        pltpu.sync_copy(tab_hbm.at[f, d], row_v)

        def bchunk_body(c, carry2):
            b0 = c * BCHUNK
            pltpu.sync_copy(idx_hbm.at[f, pl.ds(b0, BCHUNK)], idx_v)

            def bgroup(i, carry3):
                iv = idx_v[pl.ds(i * LANES, LANES)]
                out_v[pl.ds(i * LANES, LANES)] = plsc.load_gather(row_v, [iv])
                return carry3

            lax.fori_loop(0, BCHUNK // LANES, bgroup, 0)
            pltpu.sync_copy(out_v, out_hbm.at[f, d, pl.ds(b0, BCHUNK)])
            return carry2

        lax.fori_loop(0, N_BCHUNKS, bchunk_body, 0)
        return carry

    lax.fori_loop(0, PAIRS_PER_W, pair_body, 0)


_emb_call = pl.kernel(
    _emb_body,
    mesh=plsc.VectorSubcoreMesh(core_axis_name="c", subcore_axis_name="s"),
    out_type=jax.ShapeDtypeStruct((N_FIELDS, EMBED_DIM, BATCH), jnp.float32),
    scratch_types=[
        pltpu.VMEM((VOCAB,), jnp.float32),
        pltpu.VMEM((BCHUNK,), jnp.int32),
        pltpu.VMEM((BCHUNK,), jnp.float32),
    ],
    compiler_params=pltpu.CompilerParams(needs_layout_passes=False),
)


@jax.jit
def kernel(indices, tables):
    idx_t = indices.astype(jnp.int32).T
    tab_t = jnp.transpose(tables, (0, 2, 1))
    out_t = _emb_call(idx_t, tab_t)
    return jnp.transpose(out_t, (2, 0, 1))
